# baseline (device time: 53613 ns/iter reference)
import jax
import jax.numpy as jnp
from jax import lax
from jax.experimental import pallas as pl
from jax.experimental.pallas import tpu as pltpu

N_DEV = 32
CAP = 6


def kernel(x, router_W, route_idx, expert_W):
    n_tok, d = x.shape
    n_loc, _, h = expert_W.shape
    E = N_DEV * n_loc
    S = n_loc * CAP

    e = route_idx[:, 0].astype(jnp.int32)
    onehot = (e[:, None] == jnp.arange(E, dtype=jnp.int32)[None, :]).astype(
        jnp.int32
    )
    rank = jnp.sum(jnp.cumsum(onehot, axis=0) * onehot, axis=1) - 1
    keep = rank < CAP
    slot = e * CAP + rank
    dest = jnp.full((E * CAP,), -1, dtype=jnp.int32)
    dest = dest.at[jnp.where(keep, slot, E * CAP)].set(
        jnp.arange(n_tok, dtype=jnp.int32), mode="drop"
    )
    dest_tok = dest.reshape(N_DEV, S)

    def body(x_ref, ew_ref, dest_ref, out_ref, comm_ref, cx_ref, stage_ref,
             send_sems, recv_sems):
        my_pos = lax.axis_index("i")

        comm_ref[:, :, :] = jnp.zeros((n_tok, 8, h // 8), jnp.bfloat16)

        barrier_sem = pltpu.get_barrier_semaphore()
        for off in range(1, N_DEV):
            pl.semaphore_signal(
                barrier_sem,
                inc=1,
                device_id=(lax.rem(my_pos + off, N_DEV),),
                device_id_type=pl.DeviceIdType.MESH,
            )

        cx_ref[:, :] = jnp.zeros((S, d), jnp.float32)
        for j in range(S):
            tok = dest_ref[my_pos, j]

            @pl.when(tok >= 0)
            def _():
                cx_ref[pl.ds(j, 1), :] = x_ref[pl.ds(tok, 1), :]

        for k in range(n_loc):
            r = lax.dot_general(
                cx_ref[CAP * k : CAP * (k + 1), :],
                ew_ref[k],
                (((1,), (0,)), ((), ())),
                preferred_element_type=jnp.float32,
            )
            stage_ref[pl.ds(CAP * k, CAP), :, :] = r.astype(jnp.bfloat16).reshape(
                CAP, 8, h // 8
            )

        pl.semaphore_wait(barrier_sem, N_DEV - 1)

        def row_rdma(src_j, dst_tok, sem_idx, tgt):
            return pltpu.make_async_remote_copy(
                src_ref=stage_ref.at[pl.ds(src_j, 1)],
                dst_ref=comm_ref.at[pl.ds(dst_tok, 1)],
                send_sem=send_sems.at[sem_idx],
                recv_sem=recv_sems.at[my_pos],
                device_id=(tgt,),
                device_id_type=pl.DeviceIdType.MESH,
            )

        def send_j(j, carry):
            dtok = dest_ref[my_pos, j]

            @pl.when(dtok >= 0)
            def _():
                for off in range(1, N_DEV):
                    tgt = lax.rem(my_pos + off, N_DEV)
                    row_rdma(j, dtok, tgt, tgt).start()
                comm_ref[pl.ds(dtok, 1), :, :] = stage_ref[pl.ds(j, 1), :, :]

            return carry

        lax.fori_loop(0, S, send_j, 0)

        for off in range(1, N_DEV):
            origin = lax.rem(my_pos - off + N_DEV, N_DEV)

            def recv_j(j, carry, origin=origin):
                dtok = dest_ref[origin, j]

                @pl.when(dtok >= 0)
                def _():
                    pltpu.make_async_remote_copy(
                        src_ref=stage_ref.at[pl.ds(j, 1)],
                        dst_ref=comm_ref.at[pl.ds(dtok, 1)],
                        send_sem=send_sems.at[origin],
                        recv_sem=recv_sems.at[origin],
                        device_id=(origin,),
                        device_id_type=pl.DeviceIdType.MESH,
                    ).wait_recv()

                return carry

            lax.fori_loop(0, S, recv_j, 0)

        out_ref[:, :] = (
            comm_ref[:, :, :].astype(jnp.float32).reshape(n_tok, h)
        )

        def drain_j(j, carry):
            dtok = dest_ref[my_pos, j]

            @pl.when(dtok >= 0)
            def _():
                for off in range(1, N_DEV):
                    tgt = lax.rem(my_pos + off, N_DEV)
                    row_rdma(j, dtok, tgt, tgt).wait_send()

            return carry

        lax.fori_loop(0, S, drain_j, 0)

    return pl.pallas_call(
        body,
        out_shape=jax.ShapeDtypeStruct((n_tok, h), jnp.float32),
        in_specs=[
            pl.BlockSpec(memory_space=pltpu.VMEM),
            pl.BlockSpec(memory_space=pltpu.VMEM),
            pl.BlockSpec(memory_space=pltpu.SMEM),
        ],
        out_specs=pl.BlockSpec(memory_space=pltpu.VMEM),
        scratch_shapes=[
            pltpu.VMEM((n_tok, 8, h // 8), jnp.bfloat16),
            pltpu.VMEM((S, d), jnp.float32),
            pltpu.VMEM((S, 8, h // 8), jnp.bfloat16),
            pltpu.SemaphoreType.DMA((N_DEV,)),
            pltpu.SemaphoreType.DMA((N_DEV,)),
        ],
        compiler_params=pltpu.CompilerParams(collective_id=0),
    )(x, expert_W, dest_tok)


# device time: 44495 ns/iter; 1.2049x vs baseline; 1.2049x over previous
import jax
import jax.numpy as jnp
from jax import lax
from jax.experimental import pallas as pl
from jax.experimental.pallas import tpu as pltpu

N_DEV = 32
CAP = 6


def kernel(x, router_W, route_idx, expert_W):
    n_tok, d = x.shape
    n_loc, _, h = expert_W.shape
    E = N_DEV * n_loc
    S = n_loc * CAP
    hl = h // 8

    e = route_idx[:, 0].astype(jnp.int32)
    onehot = (e[:, None] == jnp.arange(E, dtype=jnp.int32)[None, :]).astype(
        jnp.int32
    )
    rank = jnp.sum(jnp.cumsum(onehot, axis=0) * onehot, axis=1) - 1
    keep = rank < CAP
    slot = e * CAP + rank
    dest = jnp.full((E * CAP,), -1, dtype=jnp.int32)
    dest = dest.at[jnp.where(keep, slot, E * CAP)].set(
        jnp.arange(n_tok, dtype=jnp.int32), mode="drop"
    )
    dest_tok = dest.reshape(N_DEV, S)

    def body(x_ref, ew_ref, dest_ref, out_ref, comm_ref, cx_ref, stage_ref,
             send_sems, recv_sems):
        my_pos = lax.axis_index("i")

        barrier_sem = pltpu.get_barrier_semaphore()
        for off in range(1, N_DEV):
            pl.semaphore_signal(
                barrier_sem,
                inc=1,
                device_id=(lax.rem(my_pos + off, N_DEV),),
                device_id_type=pl.DeviceIdType.MESH,
            )

        cx_ref[:, :] = jnp.zeros((S, d), jnp.float32)
        for j in range(S):
            tok = dest_ref[my_pos, j]

            @pl.when(tok >= 0)
            def _():
                cx_ref[pl.ds(j, 1), :] = x_ref[pl.ds(tok, 1), :]

        for k in range(n_loc):
            r = lax.dot_general(
                cx_ref[CAP * k : CAP * (k + 1), :],
                ew_ref[k],
                (((1,), (0,)), ((), ())),
                preferred_element_type=jnp.float32,
            )
            comm_ref[pl.ds(my_pos, 1), pl.ds(CAP * k, CAP), :, :] = r.astype(
                jnp.bfloat16
            ).reshape(CAP, 8, hl)[None]

        pl.semaphore_wait(barrier_sem, N_DEV - 1)

        def send_to(tgt):
            return pltpu.make_async_remote_copy(
                src_ref=comm_ref.at[pl.ds(my_pos, 1)],
                dst_ref=comm_ref.at[pl.ds(my_pos, 1)],
                send_sem=send_sems.at[tgt],
                recv_sem=recv_sems.at[my_pos],
                device_id=(tgt,),
                device_id_type=pl.DeviceIdType.MESH,
            )

        def recv_from(origin):
            return pltpu.make_async_remote_copy(
                src_ref=comm_ref.at[pl.ds(origin, 1)],
                dst_ref=comm_ref.at[pl.ds(origin, 1)],
                send_sem=send_sems.at[origin],
                recv_sem=recv_sems.at[origin],
                device_id=(origin,),
                device_id_type=pl.DeviceIdType.MESH,
            )

        for off in range(1, N_DEV):
            send_to(lax.rem(my_pos + off, N_DEV)).start()

        out_ref[:, :, :] = jnp.zeros((n_tok, 8, hl), jnp.float32)

        def scatter_chunk(s):
            stage_ref[:, :, :] = comm_ref[pl.ds(s, 1)][0].astype(jnp.float32)
            for j in range(S):
                dtok = dest_ref[s, j]

                @pl.when(dtok >= 0)
                def _():
                    out_ref[pl.ds(dtok, 1), :, :] = stage_ref[pl.ds(j, 1), :, :]

        scatter_chunk(my_pos)

        for off in range(1, N_DEV):
            origin = lax.rem(my_pos - off + N_DEV, N_DEV)
            recv_from(origin).wait_recv()
            scatter_chunk(origin)

        for off in range(1, N_DEV):
            send_to(lax.rem(my_pos + off, N_DEV)).wait_send()

    out3 = pl.pallas_call(
        body,
        out_shape=jax.ShapeDtypeStruct((n_tok, 8, hl), jnp.float32),
        in_specs=[
            pl.BlockSpec(memory_space=pltpu.VMEM),
            pl.BlockSpec(memory_space=pltpu.VMEM),
            pl.BlockSpec(memory_space=pltpu.SMEM),
        ],
        out_specs=pl.BlockSpec(memory_space=pltpu.VMEM),
        scratch_shapes=[
            pltpu.VMEM((N_DEV, S, 8, hl), jnp.bfloat16),
            pltpu.VMEM((S, d), jnp.float32),
            pltpu.VMEM((S, 8, hl), jnp.float32),
            pltpu.SemaphoreType.DMA((N_DEV,)),
            pltpu.SemaphoreType.DMA((N_DEV,)),
        ],
        compiler_params=pltpu.CompilerParams(collective_id=0),
    )(x, expert_W, dest_tok)
    return out3.reshape(n_tok, h)


# device time: 36919 ns/iter; 1.4522x vs baseline; 1.2052x over previous
import jax
import jax.numpy as jnp
from jax import lax
from jax.experimental import pallas as pl
from jax.experimental.pallas import tpu as pltpu

N_DEV = 32
CAP = 6


def kernel(x, router_W, route_idx, expert_W):
    n_tok, d = x.shape
    n_loc, _, h = expert_W.shape
    E = N_DEV * n_loc
    S = n_loc * CAP
    hl = h // 8

    e = route_idx[:, 0].astype(jnp.int32)
    onehot = (e[:, None] == jnp.arange(E, dtype=jnp.int32)[None, :]).astype(
        jnp.int32
    )
    rank = jnp.sum(jnp.cumsum(onehot, axis=0) * onehot, axis=1) - 1
    keep = rank < CAP
    slot = jnp.where(keep, e * CAP + rank, -1)
    slot_oh = (
        slot[None, :] == jnp.arange(E * CAP, dtype=jnp.int32)[:, None]
    )
    dest = (
        jnp.sum(
            jnp.where(slot_oh, jnp.arange(1, n_tok + 1, dtype=jnp.int32), 0),
            axis=1,
        )
        - 1
    )
    dest = jnp.where(dest < 0, n_tok, dest)
    dest_tok = dest.reshape(N_DEV, S)

    def body(x_ref, ew_ref, dest_ref, out_ref, comm_ref, cx_ref, stage_ref,
             send_sems, recv_sems):
        my_pos = lax.axis_index("i")

        barrier_sem = pltpu.get_barrier_semaphore()
        for off in range(1, N_DEV):
            pl.semaphore_signal(
                barrier_sem,
                inc=1,
                device_id=(lax.rem(my_pos + off, N_DEV),),
                device_id_type=pl.DeviceIdType.MESH,
            )

        for j in range(S):
            tok = dest_ref[my_pos, j]

            @pl.when(tok < n_tok)
            def _():
                cx_ref[pl.ds(j, 1), :] = x_ref[pl.ds(tok, 1), :]

        for k in range(n_loc):
            r = lax.dot_general(
                cx_ref[CAP * k : CAP * (k + 1), :],
                ew_ref[k],
                (((1,), (0,)), ((), ())),
                preferred_element_type=jnp.float32,
            )
            comm_ref[pl.ds(my_pos, 1), pl.ds(CAP * k, CAP), :, :] = r.astype(
                jnp.bfloat16
            ).reshape(CAP, 8, hl)[None]

        pl.semaphore_wait(barrier_sem, N_DEV - 1)

        def send_to(tgt):
            return pltpu.make_async_remote_copy(
                src_ref=comm_ref.at[pl.ds(my_pos, 1)],
                dst_ref=comm_ref.at[pl.ds(my_pos, 1)],
                send_sem=send_sems.at[tgt],
                recv_sem=recv_sems.at[my_pos],
                device_id=(tgt,),
                device_id_type=pl.DeviceIdType.MESH,
            )

        def recv_from(origin):
            return pltpu.make_async_remote_copy(
                src_ref=comm_ref.at[pl.ds(origin, 1)],
                dst_ref=comm_ref.at[pl.ds(origin, 1)],
                send_sem=send_sems.at[origin],
                recv_sem=recv_sems.at[origin],
                device_id=(origin,),
                device_id_type=pl.DeviceIdType.MESH,
            )

        for off in range(1, N_DEV):
            send_to(lax.rem(my_pos + off, N_DEV)).start()

        out_ref[:, :, :] = jnp.zeros((n_tok + 8, 8, hl), jnp.float32)

        def scatter_chunk(s):
            stage_ref[:, :, :] = comm_ref[pl.ds(s, 1)][0].astype(jnp.float32)
            for j in range(S):
                dtok = dest_ref[s, j]
                out_ref[pl.ds(dtok, 1), :, :] = stage_ref[pl.ds(j, 1), :, :]

        scatter_chunk(my_pos)

        for off in range(1, N_DEV):
            origin = lax.rem(my_pos - off + N_DEV, N_DEV)
            recv_from(origin).wait_recv()
            scatter_chunk(origin)

        for off in range(1, N_DEV):
            send_to(lax.rem(my_pos + off, N_DEV)).wait_send()

    out3 = pl.pallas_call(
        body,
        out_shape=jax.ShapeDtypeStruct((n_tok + 8, 8, hl), jnp.float32),
        in_specs=[
            pl.BlockSpec(memory_space=pltpu.VMEM),
            pl.BlockSpec(memory_space=pltpu.VMEM),
            pl.BlockSpec(memory_space=pltpu.SMEM),
        ],
        out_specs=pl.BlockSpec(memory_space=pltpu.VMEM),
        scratch_shapes=[
            pltpu.VMEM((N_DEV, S, 8, hl), jnp.bfloat16),
            pltpu.VMEM((S, d), jnp.float32),
            pltpu.VMEM((S, 8, hl), jnp.float32),
            pltpu.SemaphoreType.DMA((N_DEV,)),
            pltpu.SemaphoreType.DMA((N_DEV,)),
        ],
        compiler_params=pltpu.CompilerParams(collective_id=0),
    )(x, expert_W, dest_tok)
    return out3[:n_tok].reshape(n_tok, h)
